# baseline (speedup 1.0000x reference)
import jax
import jax.numpy as jnp
from jax import lax
from jax.experimental import pallas as pl
from jax.experimental.pallas import tpu as pltpu
from jax.experimental.pallas import tpu_sc as plsc

N = 16384
D_DRV, D_WK, D_TM = 16, 3, 8
D_OUT = D_DRV + D_WK + D_TM + 1

_info = plsc.get_sparse_core_info()
NC, NS, L = _info.num_cores, _info.num_subcores, _info.num_lanes
NW = NC * NS
B_W = N // NW


D_REST = D_OUT - D_DRV
CHUNKS = B_W // L


def _body(drv_idx_hbm, wk_idx_hbm, tm_idx_hbm, dist_hbm,
          wd_hbm, wk_hbm, wt_hbm, out_hbm,
          drv_idx_v, wk_idx_v, tm_idx_v, dist_v,
          drv_rows_v, wk_tab_v, tm_tab_v, rest_v, sem):
    wid = lax.axis_index("s") * NC + lax.axis_index("c")
    base = wid * B_W

    pltpu.sync_copy(drv_idx_hbm.at[pl.ds(base, B_W)], drv_idx_v)
    g1 = pltpu.async_copy(wd_hbm.at[drv_idx_v], drv_rows_v, sem)
    pltpu.sync_copy(wk_idx_hbm.at[pl.ds(base, B_W)], wk_idx_v)
    pltpu.sync_copy(tm_idx_hbm.at[pl.ds(base, B_W)], tm_idx_v)
    pltpu.sync_copy(dist_hbm.at[pl.ds(base, B_W)], dist_v)
    pltpu.sync_copy(wk_hbm, wk_tab_v)
    pltpu.sync_copy(wt_hbm, tm_tab_v)

    iota = lax.iota(jnp.int32, L)

    def chunk(i, carry):
        rows = i * L + iota
        wk16 = plsc.load_gather(wk_idx_v, [rows])
        for c in range(D_WK):
            val = plsc.load_gather(wk_tab_v, [wk16, iota * 0 + c])
            plsc.store_scatter(rest_v, [rows, iota * 0 + c], val)
        tm16 = plsc.load_gather(tm_idx_v, [rows])
        for c in range(D_TM):
            val = plsc.load_gather(tm_tab_v, [tm16, iota * 0 + c])
            plsc.store_scatter(rest_v, [rows, iota * 0 + (D_WK + c)], val)
        d16 = plsc.load_gather(dist_v, [rows])
        plsc.store_scatter(rest_v, [rows, iota * 0 + (D_REST - 1)], d16)
        return carry

    lax.fori_loop(0, CHUNKS, chunk, 0)

    rows = out_hbm.at[pl.ds(base, B_W)]
    g1.wait()
    pltpu.sync_copy(drv_rows_v, rows.at[:, pl.ds(0, D_DRV)])
    pltpu.sync_copy(rest_v, rows.at[:, pl.ds(D_DRV, D_REST)])


@jax.jit
def _run(drv_idx, wk_idx, tm_idx, dist, wd, wk, wt):
    mesh = plsc.VectorSubcoreMesh(core_axis_name="c", subcore_axis_name="s")
    f = pl.kernel(
        _body, mesh=mesh,
        compiler_params=pltpu.CompilerParams(
            needs_layout_passes=False, use_tc_tiling_on_sc=False),
        out_type=jax.ShapeDtypeStruct((N, D_OUT), jnp.float32),
        scratch_types=[
            pltpu.VMEM((B_W,), jnp.int32),
            pltpu.VMEM((B_W,), jnp.int32),
            pltpu.VMEM((B_W,), jnp.int32),
            pltpu.VMEM((B_W,), jnp.float32),
            pltpu.VMEM((B_W, D_DRV), jnp.float32),
            pltpu.VMEM((7, D_WK), jnp.float32),
            pltpu.VMEM((1440, D_TM), jnp.float32),
            pltpu.VMEM((B_W, D_REST), jnp.float32),
            pltpu.SemaphoreType.DMA,
        ],
    )
    return f(drv_idx, wk_idx, tm_idx, dist, wd, wk, wt)


def kernel(driverID, weekID, timeID, dist, W_driver, W_week, W_time):
    drv_idx = driverID.astype(jnp.int32).reshape(-1)
    wk_idx = weekID.astype(jnp.int32).reshape(-1)
    tm_idx = timeID.astype(jnp.int32).reshape(-1)
    return _run(drv_idx, wk_idx, tm_idx, dist.reshape(-1),
                W_driver, W_week, W_time)


# speedup vs baseline: 1.3351x; 1.3351x over previous
import jax
import jax.numpy as jnp
from jax import lax
from jax.experimental import pallas as pl
from jax.experimental.pallas import tpu as pltpu
from jax.experimental.pallas import tpu_sc as plsc

N = 16384
D_DRV, D_WK, D_TM = 16, 3, 8
D_OUT = D_DRV + D_WK + D_TM + 1
WK_PAD = 8

_info = plsc.get_sparse_core_info()
NC, NS, L = _info.num_cores, _info.num_subcores, _info.num_lanes
NW = NC * NS
B_W = N // NW


def _body(drv_idx_hbm, wk_idx_hbm, tm_idx_hbm, dist_hbm,
          wd_t_hbm, wk_t_hbm, wt_t_hbm, out_hbm,
          drv_idx_v, wk_idx_v, tm_idx_v, buf_v,
          wd_sp, wk_sp, wt_sp, sem):
    s = lax.axis_index("s")
    c = lax.axis_index("c")
    wid = s * NC + c
    base = wid * B_W

    pltpu.sync_copy(wd_t_hbm.at[s], wd_sp.at[s])

    @pl.when(s < D_TM)
    def _():
        pltpu.sync_copy(wt_t_hbm.at[s], wt_sp.at[s])

    @pl.when(s < D_WK)
    def _():
        pltpu.sync_copy(wk_t_hbm.at[s], wk_sp.at[s])

    pltpu.sync_copy(drv_idx_hbm.at[pl.ds(base, B_W)], drv_idx_v)
    pltpu.sync_copy(wk_idx_hbm.at[pl.ds(base, B_W)], wk_idx_v)
    pltpu.sync_copy(tm_idx_hbm.at[pl.ds(base, B_W)], tm_idx_v)
    pltpu.sync_copy(dist_hbm.at[pl.ds(base, B_W)], buf_v.at[D_OUT - 1])

    plsc.subcore_barrier()

    copies = []
    for col in range(D_DRV):
        copies.append(pltpu.async_copy(
            wd_sp.at[col].at[drv_idx_v], buf_v.at[col], sem))
    for col in range(D_WK):
        copies.append(pltpu.async_copy(
            wk_sp.at[col].at[wk_idx_v], buf_v.at[D_DRV + col], sem))
    for col in range(D_TM):
        copies.append(pltpu.async_copy(
            wt_sp.at[col].at[tm_idx_v], buf_v.at[D_DRV + D_WK + col], sem))
    for cp in copies:
        cp.wait()

    pltpu.sync_copy(buf_v, out_hbm.at[:, pl.ds(base, B_W)])


@jax.jit
def _run(drv_idx, wk_idx, tm_idx, dist, wd_t, wk_t, wt_t):
    mesh = plsc.VectorSubcoreMesh(core_axis_name="c", subcore_axis_name="s")
    f = pl.kernel(
        _body, mesh=mesh,
        compiler_params=pltpu.CompilerParams(
            needs_layout_passes=False, use_tc_tiling_on_sc=False),
        out_type=jax.ShapeDtypeStruct((D_OUT, N), jnp.float32),
        scratch_types=[
            pltpu.VMEM((B_W,), jnp.int32),
            pltpu.VMEM((B_W,), jnp.int32),
            pltpu.VMEM((B_W,), jnp.int32),
            pltpu.VMEM((D_OUT, B_W), jnp.float32),
            pltpu.VMEM_SHARED((D_DRV, 24000), jnp.float32),
            pltpu.VMEM_SHARED((D_WK, WK_PAD), jnp.float32),
            pltpu.VMEM_SHARED((D_TM, 1440), jnp.float32),
            pltpu.SemaphoreType.DMA,
        ],
    )
    return f(drv_idx, wk_idx, tm_idx, dist, wd_t, wk_t, wt_t)


def kernel(driverID, weekID, timeID, dist, W_driver, W_week, W_time):
    drv_idx = driverID.astype(jnp.int32).reshape(-1)
    wk_idx = weekID.astype(jnp.int32).reshape(-1)
    tm_idx = timeID.astype(jnp.int32).reshape(-1)
    wk_t = jnp.zeros((D_WK, WK_PAD), jnp.float32).at[:, :7].set(W_week.T)
    out_t = _run(drv_idx, wk_idx, tm_idx, dist.reshape(-1),
                 W_driver.T, wk_t, W_time.T)
    return out_t.T


# speedup vs baseline: 1.4659x; 1.0979x over previous
import jax
import jax.numpy as jnp
from jax import lax
from jax.experimental import pallas as pl
from jax.experimental.pallas import tpu as pltpu
from jax.experimental.pallas import tpu_sc as plsc

N = 16384
D_DRV, D_WK, D_TM = 16, 3, 8
D_OUT = D_DRV + D_WK + D_TM + 1

_info = plsc.get_sparse_core_info()
NC, NS, L = _info.num_cores, _info.num_subcores, _info.num_lanes
NW = NC * NS
B_W = N // NW
CHUNKS = B_W // L


def _body(drv_idx_hbm, wk_idx_hbm, tm_idx_hbm, dist_hbm,
          wd_t_hbm, wk_t_hbm, wt_t_hbm, out_hbm,
          drv_idx_v, wk_idx_v, tm_idx_v, buf_v,
          wk_tab_v, tm_tab_v, sem):
    s = lax.axis_index("s")
    c = lax.axis_index("c")
    wid = s * NC + c
    base = wid * B_W

    pltpu.sync_copy(drv_idx_hbm.at[pl.ds(base, B_W)], drv_idx_v)
    copies = [
        pltpu.async_copy(wd_t_hbm.at[col].at[drv_idx_v], buf_v.at[col], sem)
        for col in range(D_DRV)
    ]

    pltpu.sync_copy(wk_idx_hbm.at[pl.ds(base, B_W)], wk_idx_v)
    pltpu.sync_copy(tm_idx_hbm.at[pl.ds(base, B_W)], tm_idx_v)
    pltpu.sync_copy(dist_hbm.at[pl.ds(base, B_W)], buf_v.at[D_OUT - 1])
    pltpu.sync_copy(wk_t_hbm, wk_tab_v)
    pltpu.sync_copy(wt_t_hbm, tm_tab_v)

    iota = lax.iota(jnp.int32, L)

    def chunk(i, carry):
        r = i * L
        rows = r + iota
        wk16 = plsc.load_gather(wk_idx_v, [rows])
        for col in range(D_WK):
            val = plsc.load_gather(wk_tab_v, [iota * 0 + col, wk16])
            buf_v[D_DRV + col, pl.ds(r, L)] = val
        tm16 = plsc.load_gather(tm_idx_v, [rows])
        for col in range(D_TM):
            val = plsc.load_gather(tm_tab_v, [iota * 0 + col, tm16])
            buf_v[D_DRV + D_WK + col, pl.ds(r, L)] = val
        return carry

    lax.fori_loop(0, CHUNKS, chunk, 0)

    for cp in copies:
        cp.wait()

    pltpu.sync_copy(buf_v, out_hbm.at[:, pl.ds(base, B_W)])


@jax.jit
def _run(drv_idx, wk_idx, tm_idx, dist, wd_t, wk_t, wt_t):
    mesh = plsc.VectorSubcoreMesh(core_axis_name="c", subcore_axis_name="s")
    f = pl.kernel(
        _body, mesh=mesh,
        compiler_params=pltpu.CompilerParams(
            needs_layout_passes=False, use_tc_tiling_on_sc=False),
        out_type=jax.ShapeDtypeStruct((D_OUT, N), jnp.float32),
        scratch_types=[
            pltpu.VMEM((B_W,), jnp.int32),
            pltpu.VMEM((B_W,), jnp.int32),
            pltpu.VMEM((B_W,), jnp.int32),
            pltpu.VMEM((D_OUT, B_W), jnp.float32),
            pltpu.VMEM((D_WK, 7), jnp.float32),
            pltpu.VMEM((D_TM, 1440), jnp.float32),
            pltpu.SemaphoreType.DMA,
        ],
    )
    return f(drv_idx, wk_idx, tm_idx, dist, wd_t, wk_t, wt_t)


def kernel(driverID, weekID, timeID, dist, W_driver, W_week, W_time):
    drv_idx = driverID.astype(jnp.int32).reshape(-1)
    wk_idx = weekID.astype(jnp.int32).reshape(-1)
    tm_idx = timeID.astype(jnp.int32).reshape(-1)
    out_t = _run(drv_idx, wk_idx, tm_idx, dist.reshape(-1),
                 W_driver.T, W_week.T, W_time.T)
    return out_t.T


# speedup vs baseline: 1.5945x; 1.0878x over previous
import jax
import jax.numpy as jnp
from jax import lax
from jax.experimental import pallas as pl
from jax.experimental.pallas import tpu as pltpu
from jax.experimental.pallas import tpu_sc as plsc

N = 16384
D_DRV, D_WK, D_TM = 16, 3, 8
D_OUT = D_DRV + D_WK + D_TM + 1

_info = plsc.get_sparse_core_info()
NC, NS, L = _info.num_cores, _info.num_subcores, _info.num_lanes
NW = NC * NS
B_W = N // NW
CHUNKS = B_W // L


COLS_SP = 8


def _body(drv_idx_hbm, wk_idx_hbm, tm_idx_hbm, dist_hbm,
          wd_t_hbm, wk_t_hbm, wt_t_hbm, out_hbm,
          drv_idx_v, wk_idx_v, tm_idx_v, buf_v,
          wk_tab_v, tm_tab_v, wd_sp, sem, sem2):
    s = lax.axis_index("s")
    c = lax.axis_index("c")
    wid = s * NC + c
    base = wid * B_W

    pltpu.sync_copy(drv_idx_hbm.at[pl.ds(base, B_W)], drv_idx_v)

    @pl.when(s < COLS_SP)
    def _():
        pltpu.sync_copy(wd_t_hbm.at[s], wd_sp.at[s])

    plsc.subcore_barrier()
    copies = [
        pltpu.async_copy(wd_t_hbm.at[col].at[drv_idx_v], buf_v.at[col], sem)
        for col in range(COLS_SP, D_DRV)
    ]
    copies += [
        pltpu.async_copy(wd_sp.at[col].at[drv_idx_v], buf_v.at[col], sem2)
        for col in range(COLS_SP)
    ]

    pltpu.sync_copy(wk_idx_hbm.at[pl.ds(base, B_W)], wk_idx_v)
    pltpu.sync_copy(tm_idx_hbm.at[pl.ds(base, B_W)], tm_idx_v)
    pltpu.sync_copy(dist_hbm.at[pl.ds(base, B_W)], buf_v.at[D_OUT - 1])
    pltpu.sync_copy(wk_t_hbm, wk_tab_v)
    pltpu.sync_copy(wt_t_hbm, tm_tab_v)

    iota = lax.iota(jnp.int32, L)

    def chunk(i, carry):
        r = i * L
        rows = r + iota
        wk16 = plsc.load_gather(wk_idx_v, [rows])
        for col in range(D_WK):
            val = plsc.load_gather(wk_tab_v, [iota * 0 + col, wk16])
            buf_v[D_DRV + col, pl.ds(r, L)] = val
        tm16 = plsc.load_gather(tm_idx_v, [rows])
        for col in range(D_TM):
            val = plsc.load_gather(tm_tab_v, [iota * 0 + col, tm16])
            buf_v[D_DRV + D_WK + col, pl.ds(r, L)] = val
        return carry

    lax.fori_loop(0, CHUNKS, chunk, 0)

    for cp in copies:
        cp.wait()

    pltpu.sync_copy(buf_v, out_hbm.at[:, pl.ds(base, B_W)])


@jax.jit
def _run(drv_idx, wk_idx, tm_idx, dist, wd_t, wk_t, wt_t):
    mesh = plsc.VectorSubcoreMesh(core_axis_name="c", subcore_axis_name="s")
    f = pl.kernel(
        _body, mesh=mesh,
        compiler_params=pltpu.CompilerParams(
            needs_layout_passes=False, use_tc_tiling_on_sc=False),
        out_type=jax.ShapeDtypeStruct((D_OUT, N), jnp.float32),
        scratch_types=[
            pltpu.VMEM((B_W,), jnp.int32),
            pltpu.VMEM((B_W,), jnp.int32),
            pltpu.VMEM((B_W,), jnp.int32),
            pltpu.VMEM((D_OUT, B_W), jnp.float32),
            pltpu.VMEM((D_WK, 7), jnp.float32),
            pltpu.VMEM((D_TM, 1440), jnp.float32),
            pltpu.VMEM_SHARED((COLS_SP, 24000), jnp.float32),
            pltpu.SemaphoreType.DMA,
            pltpu.SemaphoreType.DMA,
        ],
    )
    return f(drv_idx, wk_idx, tm_idx, dist, wd_t, wk_t, wt_t)


def kernel(driverID, weekID, timeID, dist, W_driver, W_week, W_time):
    drv_idx = driverID.astype(jnp.int32).reshape(-1)
    wk_idx = weekID.astype(jnp.int32).reshape(-1)
    tm_idx = timeID.astype(jnp.int32).reshape(-1)
    out_t = _run(drv_idx, wk_idx, tm_idx, dist.reshape(-1),
                 W_driver.T, W_week.T, W_time.T)
    return out_t.T


# speedup vs baseline: 1.6066x; 1.0076x over previous
import jax
import jax.numpy as jnp
from jax import lax
from jax.experimental import pallas as pl
from jax.experimental.pallas import tpu as pltpu
from jax.experimental.pallas import tpu_sc as plsc

N = 16384
D_DRV, D_WK, D_TM = 16, 3, 8
D_OUT = D_DRV + D_WK + D_TM + 1

_info = plsc.get_sparse_core_info()
NC, NS, L = _info.num_cores, _info.num_subcores, _info.num_lanes
NW = NC * NS
B_W = N // NW
CHUNKS = B_W // L


COLS_SP = 11


def _body(drv_idx_hbm, wk_idx_hbm, tm_idx_hbm, dist_hbm,
          wd_t_hbm, wk_t_hbm, wt_t_hbm, out_hbm,
          drv_idx_v, wk_idx_v, tm_idx_v, buf_v,
          wk_tab_v, tm_tab_v, wd_sp, sem, sem2):
    s = lax.axis_index("s")
    c = lax.axis_index("c")
    wid = s * NC + c
    base = wid * B_W

    pltpu.sync_copy(drv_idx_hbm.at[pl.ds(base, B_W)], drv_idx_v)

    @pl.when(s < COLS_SP)
    def _():
        pltpu.sync_copy(wd_t_hbm.at[s], wd_sp.at[s])

    plsc.subcore_barrier()
    copies = [
        pltpu.async_copy(wd_t_hbm.at[col].at[drv_idx_v], buf_v.at[col], sem)
        for col in range(COLS_SP, D_DRV)
    ]
    copies += [
        pltpu.async_copy(wd_sp.at[col].at[drv_idx_v], buf_v.at[col], sem2)
        for col in range(COLS_SP)
    ]

    pltpu.sync_copy(wk_idx_hbm.at[pl.ds(base, B_W)], wk_idx_v)
    pltpu.sync_copy(tm_idx_hbm.at[pl.ds(base, B_W)], tm_idx_v)
    pltpu.sync_copy(dist_hbm.at[pl.ds(base, B_W)], buf_v.at[D_OUT - 1])
    pltpu.sync_copy(wk_t_hbm, wk_tab_v)
    pltpu.sync_copy(wt_t_hbm, tm_tab_v)

    iota = lax.iota(jnp.int32, L)

    def chunk(i, carry):
        r = i * L
        rows = r + iota
        wk16 = plsc.load_gather(wk_idx_v, [rows])
        for col in range(D_WK):
            val = plsc.load_gather(wk_tab_v, [iota * 0 + col, wk16])
            buf_v[D_DRV + col, pl.ds(r, L)] = val
        tm16 = plsc.load_gather(tm_idx_v, [rows])
        for col in range(D_TM):
            val = plsc.load_gather(tm_tab_v, [iota * 0 + col, tm16])
            buf_v[D_DRV + D_WK + col, pl.ds(r, L)] = val
        return carry

    lax.fori_loop(0, CHUNKS, chunk, 0)

    for cp in copies:
        cp.wait()

    pltpu.sync_copy(buf_v, out_hbm.at[:, pl.ds(base, B_W)])


@jax.jit
def _run(drv_idx, wk_idx, tm_idx, dist, wd_t, wk_t, wt_t):
    mesh = plsc.VectorSubcoreMesh(core_axis_name="c", subcore_axis_name="s")
    f = pl.kernel(
        _body, mesh=mesh,
        compiler_params=pltpu.CompilerParams(
            needs_layout_passes=False, use_tc_tiling_on_sc=False),
        out_type=jax.ShapeDtypeStruct((D_OUT, N), jnp.float32),
        scratch_types=[
            pltpu.VMEM((B_W,), jnp.int32),
            pltpu.VMEM((B_W,), jnp.int32),
            pltpu.VMEM((B_W,), jnp.int32),
            pltpu.VMEM((D_OUT, B_W), jnp.float32),
            pltpu.VMEM((D_WK, 7), jnp.float32),
            pltpu.VMEM((D_TM, 1440), jnp.float32),
            pltpu.VMEM_SHARED((COLS_SP, 24000), jnp.float32),
            pltpu.SemaphoreType.DMA,
            pltpu.SemaphoreType.DMA,
        ],
    )
    return f(drv_idx, wk_idx, tm_idx, dist, wd_t, wk_t, wt_t)


def kernel(driverID, weekID, timeID, dist, W_driver, W_week, W_time):
    drv_idx = driverID.astype(jnp.int32).reshape(-1)
    wk_idx = weekID.astype(jnp.int32).reshape(-1)
    tm_idx = timeID.astype(jnp.int32).reshape(-1)
    out_t = _run(drv_idx, wk_idx, tm_idx, dist.reshape(-1),
                 W_driver.T, W_week.T, W_time.T)
    return out_t.T


# speedup vs baseline: 1.6077x; 1.0007x over previous
import jax
import jax.numpy as jnp
from jax import lax
from jax.experimental import pallas as pl
from jax.experimental.pallas import tpu as pltpu
from jax.experimental.pallas import tpu_sc as plsc

N = 16384
D_DRV, D_WK, D_TM = 16, 3, 8
D_OUT = D_DRV + D_WK + D_TM + 1

_info = plsc.get_sparse_core_info()
NC, NS, L = _info.num_cores, _info.num_subcores, _info.num_lanes
NW = NC * NS
B_W = N // NW
CHUNKS = B_W // L


COLS_SP = 13


def _body(drv_idx_hbm, wk_idx_hbm, tm_idx_hbm, dist_hbm,
          wd_t_hbm, wk_t_hbm, wt_t_hbm, out_hbm,
          drv_idx_v, wk_idx_v, tm_idx_v, buf_v,
          wk_tab_v, tm_tab_v, wd_sp, sem, sem2):
    s = lax.axis_index("s")
    c = lax.axis_index("c")
    wid = s * NC + c
    base = wid * B_W

    pltpu.sync_copy(drv_idx_hbm.at[pl.ds(base, B_W)], drv_idx_v)

    @pl.when(s < COLS_SP)
    def _():
        pltpu.sync_copy(wd_t_hbm.at[s], wd_sp.at[s])

    plsc.subcore_barrier()
    copies = [
        pltpu.async_copy(wd_t_hbm.at[col].at[drv_idx_v], buf_v.at[col], sem)
        for col in range(COLS_SP, D_DRV)
    ]
    copies += [
        pltpu.async_copy(wd_sp.at[col].at[drv_idx_v], buf_v.at[col], sem2)
        for col in range(COLS_SP)
    ]

    pltpu.sync_copy(wk_idx_hbm.at[pl.ds(base, B_W)], wk_idx_v)
    pltpu.sync_copy(tm_idx_hbm.at[pl.ds(base, B_W)], tm_idx_v)
    pltpu.sync_copy(dist_hbm.at[pl.ds(base, B_W)], buf_v.at[D_OUT - 1])
    pltpu.sync_copy(wk_t_hbm, wk_tab_v)
    pltpu.sync_copy(wt_t_hbm, tm_tab_v)

    iota = lax.iota(jnp.int32, L)

    def chunk(i, carry):
        r = i * L
        rows = r + iota
        wk16 = plsc.load_gather(wk_idx_v, [rows])
        for col in range(D_WK):
            val = plsc.load_gather(wk_tab_v, [iota * 0 + col, wk16])
            buf_v[D_DRV + col, pl.ds(r, L)] = val
        tm16 = plsc.load_gather(tm_idx_v, [rows])
        for col in range(D_TM):
            val = plsc.load_gather(tm_tab_v, [iota * 0 + col, tm16])
            buf_v[D_DRV + D_WK + col, pl.ds(r, L)] = val
        return carry

    lax.fori_loop(0, CHUNKS, chunk, 0)

    for cp in copies:
        cp.wait()

    pltpu.sync_copy(buf_v, out_hbm.at[:, pl.ds(base, B_W)])


@jax.jit
def _run(drv_idx, wk_idx, tm_idx, dist, wd_t, wk_t, wt_t):
    mesh = plsc.VectorSubcoreMesh(core_axis_name="c", subcore_axis_name="s")
    f = pl.kernel(
        _body, mesh=mesh,
        compiler_params=pltpu.CompilerParams(
            needs_layout_passes=False, use_tc_tiling_on_sc=False),
        out_type=jax.ShapeDtypeStruct((D_OUT, N), jnp.float32),
        scratch_types=[
            pltpu.VMEM((B_W,), jnp.int32),
            pltpu.VMEM((B_W,), jnp.int32),
            pltpu.VMEM((B_W,), jnp.int32),
            pltpu.VMEM((D_OUT, B_W), jnp.float32),
            pltpu.VMEM((D_WK, 7), jnp.float32),
            pltpu.VMEM((D_TM, 1440), jnp.float32),
            pltpu.VMEM_SHARED((COLS_SP, 24000), jnp.float32),
            pltpu.SemaphoreType.DMA,
            pltpu.SemaphoreType.DMA,
        ],
    )
    return f(drv_idx, wk_idx, tm_idx, dist, wd_t, wk_t, wt_t)


def kernel(driverID, weekID, timeID, dist, W_driver, W_week, W_time):
    drv_idx = driverID.astype(jnp.int32).reshape(-1)
    wk_idx = weekID.astype(jnp.int32).reshape(-1)
    tm_idx = timeID.astype(jnp.int32).reshape(-1)
    out_t = _run(drv_idx, wk_idx, tm_idx, dist.reshape(-1),
                 W_driver.T, W_week.T, W_time.T)
    return out_t.T
